# baseline (speedup 1.0000x reference)
import functools

import jax
import jax.numpy as jnp
from jax import lax
from jax.experimental import pallas as pl
from jax.experimental.pallas import tpu as pltpu
from jax.experimental.pallas import tpu_sc as plsc

N = 50000
E = 800000
PERIODS = 12
HID = 32

E_PAD = 819200
TILE_E1 = E_PAD // 32
C1 = 3200
HALF_E = E_PAD // 2
C2 = 4096
BN = 2048

_mesh = plsc.VectorSubcoreMesh(core_axis_name="c", subcore_axis_name="s")
_sc_params = pltpu.CompilerParams(needs_layout_passes=False)


@functools.partial(
    pl.kernel,
    out_type=jax.ShapeDtypeStruct((32, N), jnp.float32),
    mesh=_mesh,
    compiler_params=_sc_params,
    scratch_types=[
        pltpu.VMEM((N,), jnp.float32),
        pltpu.VMEM((C1,), jnp.int32),
        pltpu.VMEM((C1,), jnp.float32),
    ],
)
def _deg_kernel(dst_hbm, w_hbm, out_hbm, acc_v, d_v, w_v):
    wid = lax.axis_index("s") * 2 + lax.axis_index("c")

    def zero_body(i, _):
        acc_v[pl.ds(i * 16, 16)] = jnp.zeros((16,), jnp.float32)
        return 0

    lax.fori_loop(0, N // 16, zero_body, 0)

    base = wid * TILE_E1

    def chunk_body(k, _):
        off = base + k * C1
        pltpu.sync_copy(dst_hbm.at[pl.ds(off, C1)], d_v)
        pltpu.sync_copy(w_hbm.at[pl.ds(off, C1)], w_v)

        for j in range(C1 // 16):
            sl = pl.ds(j * 16, 16)
            plsc.addupdate_scatter(acc_v, [d_v[sl]], w_v[sl])
        return 0

    lax.fori_loop(0, TILE_E1 // C1, chunk_body, 0)
    pltpu.sync_copy(acc_v, out_hbm.at[wid])


@functools.partial(
    pl.kernel,
    out_type=jax.ShapeDtypeStruct((24, N), jnp.float32),
    mesh=_mesh,
    compiler_params=_sc_params,
    scratch_types=[
        pltpu.VMEM((N,), jnp.float32),
        pltpu.VMEM((N,), jnp.float32),
        pltpu.VMEM((C2,), jnp.int32),
        pltpu.VMEM((C2,), jnp.int32),
        pltpu.VMEM((C2,), jnp.float32),
    ],
)
def _scatter_kernel(xsT_hbm, src_hbm, dst_hbm, w_hbm, out_hbm,
                    xf_v, acc_v, s_v, d_v, w_v):
    wid = lax.axis_index("s") * 2 + lax.axis_index("c")

    @pl.when(wid < 24)
    def _():
        f = wid // 2
        h = wid % 2
        pltpu.sync_copy(xsT_hbm.at[f], xf_v)

        def zero_body(i, _):
            acc_v[pl.ds(i * 16, 16)] = jnp.zeros((16,), jnp.float32)
            return 0

        lax.fori_loop(0, N // 16, zero_body, 0)

        base = h * HALF_E

        def chunk_body(k, _):
            off = base + k * C2
            pltpu.sync_copy(src_hbm.at[pl.ds(off, C2)], s_v)
            pltpu.sync_copy(dst_hbm.at[pl.ds(off, C2)], d_v)
            pltpu.sync_copy(w_hbm.at[pl.ds(off, C2)], w_v)

            for j in range(C2 // 16):
                sl = pl.ds(j * 16, 16)
                xv = plsc.load_gather(xf_v, [s_v[sl]])
                plsc.addupdate_scatter(acc_v, [d_v[sl]], xv * w_v[sl])
            return 0

        lax.fori_loop(0, HALF_E // C2, chunk_body, 0)
        pltpu.sync_copy(acc_v, out_hbm.at[wid])


def _mid_body(dp_ref, xT_ref, xsT_ref, dis_ref):
    deg = jnp.sum(dp_ref[...], axis=0, keepdims=True) + 1.0
    dis = lax.rsqrt(deg)
    dis = dis * (1.5 - 0.5 * deg * dis * dis)
    xs = xT_ref[...] * dis
    xsT_ref[...] = jnp.concatenate(
        [xs, jnp.zeros((16 - PERIODS, xs.shape[1]), xs.dtype)], axis=0)
    dis_ref[...] = dis


_mid_call = pl.pallas_call(
    _mid_body,
    grid=(25,),
    in_specs=[
        pl.BlockSpec((32, BN), lambda i: (0, i)),
        pl.BlockSpec((PERIODS, BN), lambda i: (0, i)),
    ],
    out_specs=[
        pl.BlockSpec((16, BN), lambda i: (0, i)),
        pl.BlockSpec((1, BN), lambda i: (0, i)),
    ],
    out_shape=[
        jax.ShapeDtypeStruct((16, N), jnp.float32),
        jax.ShapeDtypeStruct((1, N), jnp.float32),
    ],
)


def _final_body(gp_ref, xsT_ref, dis_ref, p_ref, q_ref, out_ref):
    dis = dis_ref[...]
    az = p_ref[:, 0:1]
    cz = p_ref[:, 1:2]
    ah = p_ref[:, 2:3]
    ch = p_ref[:, 3:4]
    lw = p_ref[:, 4:5]
    acc = jnp.zeros((HID, dis.shape[1]), jnp.float32)
    for t in range(PERIODS):
        g = (gp_ref[2 * t:2 * t + 1, :] + gp_ref[2 * t + 1:2 * t + 2, :]
             + xsT_ref[t:t + 1, :]) * dis
        u = az * g + cz
        v = ah * g + ch
        acc = acc + q_ref[t:t + 1, 0:1] * (jax.nn.sigmoid(-u) * jnp.tanh(v))
    h = jnp.maximum(acc, 0.0)
    out_ref[...] = jnp.sum(h * lw, axis=0, keepdims=True) + q_ref[12:13, 0:1]


_final_call = pl.pallas_call(
    _final_body,
    grid=(25,),
    in_specs=[
        pl.BlockSpec((24, BN), lambda i: (0, i)),
        pl.BlockSpec((16, BN), lambda i: (0, i)),
        pl.BlockSpec((1, BN), lambda i: (0, i)),
        pl.BlockSpec((HID, 8), lambda i: (0, 0)),
        pl.BlockSpec((16, 8), lambda i: (0, 0)),
    ],
    out_specs=pl.BlockSpec((1, BN), lambda i: (0, i)),
    out_shape=jax.ShapeDtypeStruct((1, N), jnp.float32),
)


def kernel(x, edge_index, edge_weight, att, Wz, bz, LzW, Lzb,
           Wr, br, LrW, Lrb, Wh, bh, LhW, Lhb, linW, linb):
    del Wr, br, LrW, Lrb
    src = edge_index[0].astype(jnp.int32)
    dst = edge_index[1].astype(jnp.int32)
    ew = edge_weight.astype(jnp.float32)
    pad = E_PAD - E
    src_p = jnp.concatenate([src, jnp.zeros((pad,), jnp.int32)])
    dst_p = jnp.concatenate([dst, jnp.zeros((pad,), jnp.int32)])
    w_p = jnp.concatenate([ew, jnp.zeros((pad,), jnp.float32)])
    xT = x.T

    deg_part = _deg_kernel(dst_p, w_p)
    xsT, disR = _mid_call(deg_part, xT)
    gpart = _scatter_kernel(xsT, src_p, dst_p, w_p)

    top = LzW[:HID]
    az = (Wz @ top)[0]
    cz = bz @ top + Lzb
    toph = LhW[:HID]
    ah = (Wh @ toph)[0]
    ch = bh @ toph + Lhb
    zeros = jnp.zeros((HID,), jnp.float32)
    p_arr = jnp.stack([az, cz, ah, ch, linW[:, 0], zeros, zeros, zeros], axis=1)
    probs = jax.nn.softmax(att)
    q_arr = (jnp.zeros((16, 8), jnp.float32)
             .at[:PERIODS, 0].set(probs)
             .at[12, 0].set(linb[0]))

    out_row = _final_call(gpart, xsT, disR, p_arr, q_arr)
    return out_row.reshape(N, 1)


# speedup vs baseline: 1.2945x; 1.2945x over previous
import functools

import jax
import jax.numpy as jnp
from jax import lax
from jax.experimental import pallas as pl
from jax.experimental.pallas import tpu as pltpu
from jax.experimental.pallas import tpu_sc as plsc

N = 50000
E = 800000
PERIODS = 12
HID = 32

E_PAD = 819200
TILE_E1 = E_PAD // 32
C1 = 3200
HALF_E = E_PAD // 2
C2 = 4096
BN = 2048

_mesh = plsc.VectorSubcoreMesh(core_axis_name="c", subcore_axis_name="s")
_sc_params = pltpu.CompilerParams(needs_layout_passes=False)


@functools.partial(
    pl.kernel,
    out_type=jax.ShapeDtypeStruct((32, N), jnp.float32),
    mesh=_mesh,
    compiler_params=_sc_params,
    scratch_types=[
        pltpu.VMEM((N,), jnp.float32),
        pltpu.VMEM((C1,), jnp.int32),
        pltpu.VMEM((C1,), jnp.float32),
    ],
)
def _deg_kernel(dst_hbm, w_hbm, out_hbm, acc_v, d_v, w_v):
    wid = lax.axis_index("s") * 2 + lax.axis_index("c")

    def zero_body(i, _):
        acc_v[pl.ds(i * 16, 16)] = jnp.zeros((16,), jnp.float32)
        return 0

    lax.fori_loop(0, N // 16, zero_body, 0)

    base = wid * TILE_E1

    def chunk_body(k, _):
        off = base + k * C1
        pltpu.sync_copy(dst_hbm.at[pl.ds(off, C1)], d_v)
        pltpu.sync_copy(w_hbm.at[pl.ds(off, C1)], w_v)

        for j in range(C1 // 16):
            sl = pl.ds(j * 16, 16)
            plsc.addupdate_scatter(acc_v, [d_v[sl]], w_v[sl])
        return 0

    lax.fori_loop(0, TILE_E1 // C1, chunk_body, 0)
    pltpu.sync_copy(acc_v, out_hbm.at[wid])


@functools.partial(
    pl.kernel,
    out_type=jax.ShapeDtypeStruct((24, N), jnp.float32),
    mesh=_mesh,
    compiler_params=_sc_params,
    scratch_types=[
        pltpu.VMEM((N,), jnp.float32),
        pltpu.VMEM((N,), jnp.float32),
        pltpu.VMEM((2, C2), jnp.int32),
        pltpu.VMEM((2, C2), jnp.int32),
        pltpu.VMEM((2, C2), jnp.float32),
        pltpu.SemaphoreType.DMA,
        pltpu.SemaphoreType.DMA,
    ],
)
def _scatter_kernel(xsT_hbm, src_hbm, dst_hbm, w_hbm, out_hbm,
                    xf_v, acc_v, s_v, d_v, w_v, sem0, sem1):
    wid = lax.axis_index("s") * 2 + lax.axis_index("c")

    @pl.when(wid < 24)
    def _():
        f = wid // 2
        h = wid % 2
        pltpu.sync_copy(xsT_hbm.at[f], xf_v)

        def zero_body(i, _):
            acc_v[pl.ds(i * 16, 16)] = jnp.zeros((16,), jnp.float32)
            return 0

        lax.fori_loop(0, N // 16, zero_body, 0)

        base = h * HALF_E
        sems = (sem0, sem1)

        def issue(b, off):
            pltpu.async_copy(src_hbm.at[pl.ds(off, C2)], s_v.at[b], sems[b])
            pltpu.async_copy(dst_hbm.at[pl.ds(off, C2)], d_v.at[b], sems[b])
            pltpu.async_copy(w_hbm.at[pl.ds(off, C2)], w_v.at[b], sems[b])

        def drain(b):
            z = pl.ds(0, C2)
            pltpu.make_async_copy(src_hbm.at[z], s_v.at[b], sems[b]).wait()
            pltpu.make_async_copy(dst_hbm.at[z], d_v.at[b], sems[b]).wait()
            pltpu.make_async_copy(w_hbm.at[z], w_v.at[b], sems[b]).wait()

        def process(b):
            for j in range(C2 // 16):
                sl = pl.ds(j * 16, 16)
                xv = plsc.load_gather(xf_v, [s_v[b, sl]])
                plsc.addupdate_scatter(acc_v, [d_v[b, sl]], xv * w_v[b, sl])

        n_pairs = HALF_E // C2 // 2
        issue(0, base)

        def pair_body(k2, _):
            off0 = base + (2 * k2) * C2
            issue(1, off0 + C2)
            drain(0)
            process(0)

            @pl.when(k2 < n_pairs - 1)
            def _():
                issue(0, off0 + 2 * C2)

            drain(1)
            process(1)
            return 0

        lax.fori_loop(0, n_pairs, pair_body, 0)
        pltpu.sync_copy(acc_v, out_hbm.at[wid])


def _mid_body(dp_ref, xT_ref, xsT_ref, dis_ref):
    deg = jnp.sum(dp_ref[...], axis=0, keepdims=True) + 1.0
    dis = lax.rsqrt(deg)
    dis = dis * (1.5 - 0.5 * deg * dis * dis)
    xs = xT_ref[...] * dis
    xsT_ref[...] = jnp.concatenate(
        [xs, jnp.zeros((16 - PERIODS, xs.shape[1]), xs.dtype)], axis=0)
    dis_ref[...] = dis


_mid_call = pl.pallas_call(
    _mid_body,
    grid=(25,),
    in_specs=[
        pl.BlockSpec((32, BN), lambda i: (0, i)),
        pl.BlockSpec((PERIODS, BN), lambda i: (0, i)),
    ],
    out_specs=[
        pl.BlockSpec((16, BN), lambda i: (0, i)),
        pl.BlockSpec((1, BN), lambda i: (0, i)),
    ],
    out_shape=[
        jax.ShapeDtypeStruct((16, N), jnp.float32),
        jax.ShapeDtypeStruct((1, N), jnp.float32),
    ],
)


def _final_body(gp_ref, xsT_ref, dis_ref, p_ref, q_ref, out_ref):
    dis = dis_ref[...]
    az = p_ref[:, 0:1]
    cz = p_ref[:, 1:2]
    ah = p_ref[:, 2:3]
    ch = p_ref[:, 3:4]
    lw = p_ref[:, 4:5]
    acc = jnp.zeros((HID, dis.shape[1]), jnp.float32)
    for t in range(PERIODS):
        g = (gp_ref[2 * t:2 * t + 1, :] + gp_ref[2 * t + 1:2 * t + 2, :]
             + xsT_ref[t:t + 1, :]) * dis
        u = az * g + cz
        v = ah * g + ch
        acc = acc + q_ref[t:t + 1, 0:1] * (jax.nn.sigmoid(-u) * jnp.tanh(v))
    h = jnp.maximum(acc, 0.0)
    out_ref[...] = jnp.sum(h * lw, axis=0, keepdims=True) + q_ref[12:13, 0:1]


_final_call = pl.pallas_call(
    _final_body,
    grid=(25,),
    in_specs=[
        pl.BlockSpec((24, BN), lambda i: (0, i)),
        pl.BlockSpec((16, BN), lambda i: (0, i)),
        pl.BlockSpec((1, BN), lambda i: (0, i)),
        pl.BlockSpec((HID, 8), lambda i: (0, 0)),
        pl.BlockSpec((16, 8), lambda i: (0, 0)),
    ],
    out_specs=pl.BlockSpec((1, BN), lambda i: (0, i)),
    out_shape=jax.ShapeDtypeStruct((1, N), jnp.float32),
)


def kernel(x, edge_index, edge_weight, att, Wz, bz, LzW, Lzb,
           Wr, br, LrW, Lrb, Wh, bh, LhW, Lhb, linW, linb):
    del Wr, br, LrW, Lrb
    src = edge_index[0].astype(jnp.int32)
    dst = edge_index[1].astype(jnp.int32)
    ew = edge_weight.astype(jnp.float32)
    pad = E_PAD - E
    src_p = jnp.concatenate([src, jnp.zeros((pad,), jnp.int32)])
    dst_p = jnp.concatenate([dst, jnp.zeros((pad,), jnp.int32)])
    w_p = jnp.concatenate([ew, jnp.zeros((pad,), jnp.float32)])
    xT = x.T

    deg_part = _deg_kernel(dst_p, w_p)
    xsT, disR = _mid_call(deg_part, xT)
    gpart = _scatter_kernel(xsT, src_p, dst_p, w_p)

    top = LzW[:HID]
    az = (Wz @ top)[0]
    cz = bz @ top + Lzb
    toph = LhW[:HID]
    ah = (Wh @ toph)[0]
    ch = bh @ toph + Lhb
    zeros = jnp.zeros((HID,), jnp.float32)
    p_arr = jnp.stack([az, cz, ah, ch, linW[:, 0], zeros, zeros, zeros], axis=1)
    probs = jax.nn.softmax(att)
    q_arr = (jnp.zeros((16, 8), jnp.float32)
             .at[:PERIODS, 0].set(probs)
             .at[12, 0].set(linb[0]))

    out_row = _final_call(gpart, xsT, disR, p_arr, q_arr)
    return out_row.reshape(N, 1)


# speedup vs baseline: 2.0072x; 1.5506x over previous
import functools

import jax
import jax.numpy as jnp
from jax import lax
from jax.experimental import pallas as pl
from jax.experimental.pallas import tpu as pltpu
from jax.experimental.pallas import tpu_sc as plsc

N = 50000
E = 800000
PERIODS = 12
HID = 32

E_PAD = 819200
TILE_E1 = E_PAD // 32
C1 = 3200
HALF_E = E_PAD // 2
C2 = 4096
BN = 2048

_mesh = plsc.VectorSubcoreMesh(core_axis_name="c", subcore_axis_name="s")
_sc_params = pltpu.CompilerParams(needs_layout_passes=False)


@functools.partial(
    pl.kernel,
    out_type=jax.ShapeDtypeStruct((32, N), jnp.float32),
    mesh=_mesh,
    compiler_params=_sc_params,
    scratch_types=[
        pltpu.VMEM((N,), jnp.float32),
        pltpu.VMEM((C1,), jnp.int32),
        pltpu.VMEM((C1,), jnp.float32),
    ],
)
def _deg_kernel(dst_hbm, w_hbm, out_hbm, acc_v, d_v, w_v):
    wid = lax.axis_index("s") * 2 + lax.axis_index("c")

    def zero_body(i, _):
        acc_v[pl.ds(i * 16, 16)] = jnp.zeros((16,), jnp.float32)
        return 0

    lax.fori_loop(0, N // 16, zero_body, 0)

    base = wid * TILE_E1

    def chunk_body(k, _):
        off = base + k * C1
        pltpu.sync_copy(dst_hbm.at[pl.ds(off, C1)], d_v)
        pltpu.sync_copy(w_hbm.at[pl.ds(off, C1)], w_v)

        for j in range(C1 // 16):
            sl = pl.ds(j * 16, 16)
            plsc.addupdate_scatter(acc_v, [d_v[sl]], w_v[sl])
        return 0

    lax.fori_loop(0, TILE_E1 // C1, chunk_body, 0)
    pltpu.sync_copy(acc_v, out_hbm.at[wid])


@functools.partial(
    pl.kernel,
    out_type=jax.ShapeDtypeStruct((24, N), jnp.float32),
    mesh=_mesh,
    compiler_params=_sc_params,
    scratch_types=[
        pltpu.VMEM((N,), jnp.float32),
        pltpu.VMEM((N,), jnp.float32),
        pltpu.VMEM((2, C2), jnp.int32),
        pltpu.VMEM((2, C2), jnp.int32),
        pltpu.VMEM((2, C2), jnp.float32),
        pltpu.SemaphoreType.DMA,
        pltpu.SemaphoreType.DMA,
    ],
)
def _scatter_kernel(xsT_hbm, src_hbm, dst_hbm, w_hbm, out_hbm,
                    xf_v, acc_v, s_v, d_v, w_v, sem0, sem1):
    wid = lax.axis_index("s") * 2 + lax.axis_index("c")

    @pl.when(wid < 24)
    def _():
        f = wid // 2
        h = wid % 2
        pltpu.sync_copy(xsT_hbm.at[f], xf_v)

        def zero_body(i, _):
            acc_v[pl.ds(i * 16, 16)] = jnp.zeros((16,), jnp.float32)
            return 0

        lax.fori_loop(0, N // 16, zero_body, 0)

        base = h * HALF_E
        sems = (sem0, sem1)

        def issue(b, off):
            pltpu.async_copy(src_hbm.at[pl.ds(off, C2)], s_v.at[b], sems[b])
            pltpu.async_copy(dst_hbm.at[pl.ds(off, C2)], d_v.at[b], sems[b])
            pltpu.async_copy(w_hbm.at[pl.ds(off, C2)], w_v.at[b], sems[b])

        def drain(b):
            z = pl.ds(0, C2)
            pltpu.make_async_copy(src_hbm.at[z], s_v.at[b], sems[b]).wait()
            pltpu.make_async_copy(dst_hbm.at[z], d_v.at[b], sems[b]).wait()
            pltpu.make_async_copy(w_hbm.at[z], w_v.at[b], sems[b]).wait()

        def process(b):
            @plsc.parallel_loop(0, C2 // 16, 1, unroll=8)
            def _(j):
                sl = pl.ds(j * 16, 16)
                xv = plsc.load_gather(xf_v, [s_v[b, sl]])
                plsc.addupdate_scatter(acc_v, [d_v[b, sl]], xv * w_v[b, sl])

        n_pairs = HALF_E // C2 // 2
        issue(0, base)

        def pair_body(k2, _):
            off0 = base + (2 * k2) * C2
            issue(1, off0 + C2)
            drain(0)
            process(0)

            @pl.when(k2 < n_pairs - 1)
            def _():
                issue(0, off0 + 2 * C2)

            drain(1)
            process(1)
            return 0

        lax.fori_loop(0, n_pairs, pair_body, 0)
        pltpu.sync_copy(acc_v, out_hbm.at[wid])


def _mid_body(dp_ref, xT_ref, xsT_ref, dis_ref):
    deg = jnp.sum(dp_ref[...], axis=0, keepdims=True) + 1.0
    dis = lax.rsqrt(deg)
    dis = dis * (1.5 - 0.5 * deg * dis * dis)
    xs = xT_ref[...] * dis
    xsT_ref[...] = jnp.concatenate(
        [xs, jnp.zeros((16 - PERIODS, xs.shape[1]), xs.dtype)], axis=0)
    dis_ref[...] = dis


_mid_call = pl.pallas_call(
    _mid_body,
    grid=(25,),
    in_specs=[
        pl.BlockSpec((32, BN), lambda i: (0, i)),
        pl.BlockSpec((PERIODS, BN), lambda i: (0, i)),
    ],
    out_specs=[
        pl.BlockSpec((16, BN), lambda i: (0, i)),
        pl.BlockSpec((1, BN), lambda i: (0, i)),
    ],
    out_shape=[
        jax.ShapeDtypeStruct((16, N), jnp.float32),
        jax.ShapeDtypeStruct((1, N), jnp.float32),
    ],
)


def _final_body(gp_ref, xsT_ref, dis_ref, p_ref, q_ref, out_ref):
    dis = dis_ref[...]
    az = p_ref[:, 0:1]
    cz = p_ref[:, 1:2]
    ah = p_ref[:, 2:3]
    ch = p_ref[:, 3:4]
    lw = p_ref[:, 4:5]
    acc = jnp.zeros((HID, dis.shape[1]), jnp.float32)
    for t in range(PERIODS):
        g = (gp_ref[2 * t:2 * t + 1, :] + gp_ref[2 * t + 1:2 * t + 2, :]
             + xsT_ref[t:t + 1, :]) * dis
        u = az * g + cz
        v = ah * g + ch
        acc = acc + q_ref[t:t + 1, 0:1] * (jax.nn.sigmoid(-u) * jnp.tanh(v))
    h = jnp.maximum(acc, 0.0)
    out_ref[...] = jnp.sum(h * lw, axis=0, keepdims=True) + q_ref[12:13, 0:1]


_final_call = pl.pallas_call(
    _final_body,
    grid=(25,),
    in_specs=[
        pl.BlockSpec((24, BN), lambda i: (0, i)),
        pl.BlockSpec((16, BN), lambda i: (0, i)),
        pl.BlockSpec((1, BN), lambda i: (0, i)),
        pl.BlockSpec((HID, 8), lambda i: (0, 0)),
        pl.BlockSpec((16, 8), lambda i: (0, 0)),
    ],
    out_specs=pl.BlockSpec((1, BN), lambda i: (0, i)),
    out_shape=jax.ShapeDtypeStruct((1, N), jnp.float32),
)


def kernel(x, edge_index, edge_weight, att, Wz, bz, LzW, Lzb,
           Wr, br, LrW, Lrb, Wh, bh, LhW, Lhb, linW, linb):
    del Wr, br, LrW, Lrb
    src = edge_index[0].astype(jnp.int32)
    dst = edge_index[1].astype(jnp.int32)
    ew = edge_weight.astype(jnp.float32)
    pad = E_PAD - E
    src_p = jnp.concatenate([src, jnp.zeros((pad,), jnp.int32)])
    dst_p = jnp.concatenate([dst, jnp.zeros((pad,), jnp.int32)])
    w_p = jnp.concatenate([ew, jnp.zeros((pad,), jnp.float32)])
    xT = x.T

    deg_part = _deg_kernel(dst_p, w_p)
    xsT, disR = _mid_call(deg_part, xT)
    gpart = _scatter_kernel(xsT, src_p, dst_p, w_p)

    top = LzW[:HID]
    az = (Wz @ top)[0]
    cz = bz @ top + Lzb
    toph = LhW[:HID]
    ah = (Wh @ toph)[0]
    ch = bh @ toph + Lhb
    zeros = jnp.zeros((HID,), jnp.float32)
    p_arr = jnp.stack([az, cz, ah, ch, linW[:, 0], zeros, zeros, zeros], axis=1)
    probs = jax.nn.softmax(att)
    q_arr = (jnp.zeros((16, 8), jnp.float32)
             .at[:PERIODS, 0].set(probs)
             .at[12, 0].set(linb[0]))

    out_row = _final_call(gpart, xsT, disR, p_arr, q_arr)
    return out_row.reshape(N, 1)


# speedup vs baseline: 2.0798x; 1.0361x over previous
import functools

import jax
import jax.numpy as jnp
from jax import lax
from jax.experimental import pallas as pl
from jax.experimental.pallas import tpu as pltpu
from jax.experimental.pallas import tpu_sc as plsc

N = 50000
E = 800000
PERIODS = 12
HID = 32

E_PAD = 819200
TILE_E1 = E_PAD // 32
C1 = 3200
HALF_E = E_PAD // 2
C2 = 4096
BN = 2048

_mesh = plsc.VectorSubcoreMesh(core_axis_name="c", subcore_axis_name="s")
_sc_params = pltpu.CompilerParams(needs_layout_passes=False)


@functools.partial(
    pl.kernel,
    out_type=jax.ShapeDtypeStruct((32, N), jnp.float32),
    mesh=_mesh,
    compiler_params=_sc_params,
    scratch_types=[
        pltpu.VMEM((N,), jnp.float32),
        pltpu.VMEM((2, C1), jnp.int32),
        pltpu.VMEM((2, C1), jnp.float32),
        pltpu.SemaphoreType.DMA,
        pltpu.SemaphoreType.DMA,
    ],
)
def _deg_kernel(dst_hbm, w_hbm, out_hbm, acc_v, d_v, w_v, sem0, sem1):
    wid = lax.axis_index("s") * 2 + lax.axis_index("c")

    def zero_body(i, _):
        acc_v[pl.ds(i * 16, 16)] = jnp.zeros((16,), jnp.float32)
        return 0

    lax.fori_loop(0, N // 16, zero_body, 0)

    base = wid * TILE_E1
    sems = (sem0, sem1)

    def issue(b, off):
        pltpu.async_copy(dst_hbm.at[pl.ds(off, C1)], d_v.at[b], sems[b])
        pltpu.async_copy(w_hbm.at[pl.ds(off, C1)], w_v.at[b], sems[b])

    def drain(b):
        z = pl.ds(0, C1)
        pltpu.make_async_copy(dst_hbm.at[z], d_v.at[b], sems[b]).wait()
        pltpu.make_async_copy(w_hbm.at[z], w_v.at[b], sems[b]).wait()

    def process(b):
        @plsc.parallel_loop(0, C1 // 16, 1, unroll=8)
        def _(j):
            sl = pl.ds(j * 16, 16)
            plsc.addupdate_scatter(acc_v, [d_v[b, sl]], w_v[b, sl])

    n_pairs = TILE_E1 // C1 // 2
    issue(0, base)

    def pair_body(k2, _):
        off0 = base + (2 * k2) * C1
        issue(1, off0 + C1)
        drain(0)
        process(0)

        @pl.when(k2 < n_pairs - 1)
        def _():
            issue(0, off0 + 2 * C1)

        drain(1)
        process(1)
        return 0

    lax.fori_loop(0, n_pairs, pair_body, 0)
    pltpu.sync_copy(acc_v, out_hbm.at[wid])


@functools.partial(
    pl.kernel,
    out_type=jax.ShapeDtypeStruct((24, N), jnp.float32),
    mesh=_mesh,
    compiler_params=_sc_params,
    scratch_types=[
        pltpu.VMEM((N,), jnp.float32),
        pltpu.VMEM((N,), jnp.float32),
        pltpu.VMEM((2, C2), jnp.int32),
        pltpu.VMEM((2, C2), jnp.int32),
        pltpu.VMEM((2, C2), jnp.float32),
        pltpu.SemaphoreType.DMA,
        pltpu.SemaphoreType.DMA,
    ],
)
def _scatter_kernel(xsT_hbm, src_hbm, dst_hbm, w_hbm, out_hbm,
                    xf_v, acc_v, s_v, d_v, w_v, sem0, sem1):
    wid = lax.axis_index("s") * 2 + lax.axis_index("c")

    @pl.when(wid < 24)
    def _():
        f = wid // 2
        h = wid % 2
        pltpu.sync_copy(xsT_hbm.at[f], xf_v)

        def zero_body(i, _):
            acc_v[pl.ds(i * 16, 16)] = jnp.zeros((16,), jnp.float32)
            return 0

        lax.fori_loop(0, N // 16, zero_body, 0)

        base = h * HALF_E
        sems = (sem0, sem1)

        def issue(b, off):
            pltpu.async_copy(src_hbm.at[pl.ds(off, C2)], s_v.at[b], sems[b])
            pltpu.async_copy(dst_hbm.at[pl.ds(off, C2)], d_v.at[b], sems[b])
            pltpu.async_copy(w_hbm.at[pl.ds(off, C2)], w_v.at[b], sems[b])

        def drain(b):
            z = pl.ds(0, C2)
            pltpu.make_async_copy(src_hbm.at[z], s_v.at[b], sems[b]).wait()
            pltpu.make_async_copy(dst_hbm.at[z], d_v.at[b], sems[b]).wait()
            pltpu.make_async_copy(w_hbm.at[z], w_v.at[b], sems[b]).wait()

        def process(b):
            @plsc.parallel_loop(0, C2 // 16, 1, unroll=16)
            def _(j):
                sl = pl.ds(j * 16, 16)
                xv = plsc.load_gather(xf_v, [s_v[b, sl]])
                plsc.addupdate_scatter(acc_v, [d_v[b, sl]], xv * w_v[b, sl])

        n_pairs = HALF_E // C2 // 2
        issue(0, base)

        def pair_body(k2, _):
            off0 = base + (2 * k2) * C2
            issue(1, off0 + C2)
            drain(0)
            process(0)

            @pl.when(k2 < n_pairs - 1)
            def _():
                issue(0, off0 + 2 * C2)

            drain(1)
            process(1)
            return 0

        lax.fori_loop(0, n_pairs, pair_body, 0)
        pltpu.sync_copy(acc_v, out_hbm.at[wid])


def _mid_body(dp_ref, xT_ref, xsT_ref, dis_ref):
    deg = jnp.sum(dp_ref[...], axis=0, keepdims=True) + 1.0
    dis = lax.rsqrt(deg)
    dis = dis * (1.5 - 0.5 * deg * dis * dis)
    xs = xT_ref[...] * dis
    xsT_ref[...] = jnp.concatenate(
        [xs, jnp.zeros((16 - PERIODS, xs.shape[1]), xs.dtype)], axis=0)
    dis_ref[...] = dis


_mid_call = pl.pallas_call(
    _mid_body,
    grid=(25,),
    in_specs=[
        pl.BlockSpec((32, BN), lambda i: (0, i)),
        pl.BlockSpec((PERIODS, BN), lambda i: (0, i)),
    ],
    out_specs=[
        pl.BlockSpec((16, BN), lambda i: (0, i)),
        pl.BlockSpec((1, BN), lambda i: (0, i)),
    ],
    out_shape=[
        jax.ShapeDtypeStruct((16, N), jnp.float32),
        jax.ShapeDtypeStruct((1, N), jnp.float32),
    ],
)


def _final_body(gp_ref, xsT_ref, dis_ref, p_ref, q_ref, out_ref):
    dis = dis_ref[...]
    az = p_ref[:, 0:1]
    cz = p_ref[:, 1:2]
    ah = p_ref[:, 2:3]
    ch = p_ref[:, 3:4]
    lw = p_ref[:, 4:5]
    acc = jnp.zeros((HID, dis.shape[1]), jnp.float32)
    for t in range(PERIODS):
        g = (gp_ref[2 * t:2 * t + 1, :] + gp_ref[2 * t + 1:2 * t + 2, :]
             + xsT_ref[t:t + 1, :]) * dis
        u = az * g + cz
        v = ah * g + ch
        acc = acc + q_ref[t:t + 1, 0:1] * (jax.nn.sigmoid(-u) * jnp.tanh(v))
    h = jnp.maximum(acc, 0.0)
    out_ref[...] = jnp.sum(h * lw, axis=0, keepdims=True) + q_ref[12:13, 0:1]


_final_call = pl.pallas_call(
    _final_body,
    grid=(25,),
    in_specs=[
        pl.BlockSpec((24, BN), lambda i: (0, i)),
        pl.BlockSpec((16, BN), lambda i: (0, i)),
        pl.BlockSpec((1, BN), lambda i: (0, i)),
        pl.BlockSpec((HID, 8), lambda i: (0, 0)),
        pl.BlockSpec((16, 8), lambda i: (0, 0)),
    ],
    out_specs=pl.BlockSpec((1, BN), lambda i: (0, i)),
    out_shape=jax.ShapeDtypeStruct((1, N), jnp.float32),
)


def kernel(x, edge_index, edge_weight, att, Wz, bz, LzW, Lzb,
           Wr, br, LrW, Lrb, Wh, bh, LhW, Lhb, linW, linb):
    del Wr, br, LrW, Lrb
    src = edge_index[0].astype(jnp.int32)
    dst = edge_index[1].astype(jnp.int32)
    ew = edge_weight.astype(jnp.float32)
    pad = E_PAD - E
    src_p = jnp.concatenate([src, jnp.zeros((pad,), jnp.int32)])
    dst_p = jnp.concatenate([dst, jnp.zeros((pad,), jnp.int32)])
    w_p = jnp.concatenate([ew, jnp.zeros((pad,), jnp.float32)])
    xT = x.T

    deg_part = _deg_kernel(dst_p, w_p)
    xsT, disR = _mid_call(deg_part, xT)
    gpart = _scatter_kernel(xsT, src_p, dst_p, w_p)

    top = LzW[:HID]
    az = (Wz @ top)[0]
    cz = bz @ top + Lzb
    toph = LhW[:HID]
    ah = (Wh @ toph)[0]
    ch = bh @ toph + Lhb
    zeros = jnp.zeros((HID,), jnp.float32)
    p_arr = jnp.stack([az, cz, ah, ch, linW[:, 0], zeros, zeros, zeros], axis=1)
    probs = jax.nn.softmax(att)
    q_arr = (jnp.zeros((16, 8), jnp.float32)
             .at[:PERIODS, 0].set(probs)
             .at[12, 0].set(linb[0]))

    out_row = _final_call(gpart, xsT, disR, p_arr, q_arr)
    return out_row.reshape(N, 1)


# speedup vs baseline: 2.2631x; 1.0882x over previous
import functools

import jax
import jax.numpy as jnp
from jax import lax
from jax.experimental import pallas as pl
from jax.experimental.pallas import tpu as pltpu
from jax.experimental.pallas import tpu_sc as plsc

N = 50000
E = 800000
PERIODS = 12
HID = 32

E_PAD = 819200
TILE_E1 = E_PAD // 32
C1 = 3200
HALF_E = E_PAD // 2
C2 = 4096
BN = 2048

_mesh = plsc.VectorSubcoreMesh(core_axis_name="c", subcore_axis_name="s")
_sc_params = pltpu.CompilerParams(needs_layout_passes=False)


@functools.partial(
    pl.kernel,
    out_type=jax.ShapeDtypeStruct((32, N), jnp.float32),
    mesh=_mesh,
    compiler_params=_sc_params,
    scratch_types=[
        pltpu.VMEM((N,), jnp.float32),
        pltpu.VMEM((2, C1), jnp.int32),
        pltpu.VMEM((2, C1), jnp.float32),
        pltpu.SemaphoreType.DMA,
        pltpu.SemaphoreType.DMA,
    ],
)
def _deg_kernel(pk_hbm, w_hbm, out_hbm, acc_v, d_v, w_v, sem0, sem1):
    wid = lax.axis_index("s") * 2 + lax.axis_index("c")

    def zero_body(i, _):
        acc_v[pl.ds(i * 16, 16)] = jnp.zeros((16,), jnp.float32)
        return 0

    lax.fori_loop(0, N // 16, zero_body, 0)

    base = wid * TILE_E1
    sems = (sem0, sem1)

    def issue(b, off):
        pltpu.async_copy(pk_hbm.at[pl.ds(off, C1)], d_v.at[b], sems[b])
        pltpu.async_copy(w_hbm.at[pl.ds(off, C1)], w_v.at[b], sems[b])

    def drain(b):
        z = pl.ds(0, C1)
        pltpu.make_async_copy(pk_hbm.at[z], d_v.at[b], sems[b]).wait()
        pltpu.make_async_copy(w_hbm.at[z], w_v.at[b], sems[b]).wait()

    def process(b):
        @plsc.parallel_loop(0, C1 // 16, 1, unroll=8)
        def _(j):
            sl = pl.ds(j * 16, 16)
            d_idx = lax.shift_right_logical(d_v[b, sl], 16)
            plsc.addupdate_scatter(acc_v, [d_idx], w_v[b, sl])

    n_pairs = TILE_E1 // C1 // 2
    issue(0, base)

    def pair_body(k2, _):
        off0 = base + (2 * k2) * C1
        issue(1, off0 + C1)
        drain(0)
        process(0)

        @pl.when(k2 < n_pairs - 1)
        def _():
            issue(0, off0 + 2 * C1)

        drain(1)
        process(1)
        return 0

    lax.fori_loop(0, n_pairs, pair_body, 0)
    pltpu.sync_copy(acc_v, out_hbm.at[wid])


@functools.partial(
    pl.kernel,
    out_type=jax.ShapeDtypeStruct((24, N), jnp.float32),
    mesh=_mesh,
    compiler_params=_sc_params,
    scratch_types=[
        pltpu.VMEM((N,), jnp.float32),
        pltpu.VMEM((N,), jnp.float32),
        pltpu.VMEM((2, C2), jnp.int32),
        pltpu.VMEM((2, C2), jnp.float32),
        pltpu.SemaphoreType.DMA,
        pltpu.SemaphoreType.DMA,
    ],
)
def _scatter_kernel(xsT_hbm, pk_hbm, w_hbm, out_hbm,
                    xf_v, acc_v, p_v, w_v, sem0, sem1):
    wid = lax.axis_index("s") * 2 + lax.axis_index("c")

    @pl.when(wid < 24)
    def _():
        f = wid // 2
        h = wid % 2
        pltpu.sync_copy(xsT_hbm.at[f], xf_v)

        def zero_body(i, _):
            acc_v[pl.ds(i * 16, 16)] = jnp.zeros((16,), jnp.float32)
            return 0

        lax.fori_loop(0, N // 16, zero_body, 0)

        base = h * HALF_E
        sems = (sem0, sem1)

        def issue(b, off):
            pltpu.async_copy(pk_hbm.at[pl.ds(off, C2)], p_v.at[b], sems[b])
            pltpu.async_copy(w_hbm.at[pl.ds(off, C2)], w_v.at[b], sems[b])

        def drain(b):
            z = pl.ds(0, C2)
            pltpu.make_async_copy(pk_hbm.at[z], p_v.at[b], sems[b]).wait()
            pltpu.make_async_copy(w_hbm.at[z], w_v.at[b], sems[b]).wait()

        def process(b):
            @plsc.parallel_loop(0, C2 // 16, 1, unroll=16)
            def _(j):
                sl = pl.ds(j * 16, 16)
                pk = p_v[b, sl]
                s_idx = lax.bitwise_and(pk, jnp.int32(0xFFFF))
                d_idx = lax.shift_right_logical(pk, 16)
                xv = plsc.load_gather(xf_v, [s_idx])
                plsc.addupdate_scatter(acc_v, [d_idx], xv * w_v[b, sl])

        n_pairs = HALF_E // C2 // 2
        issue(0, base)

        def pair_body(k2, _):
            off0 = base + (2 * k2) * C2
            issue(1, off0 + C2)
            drain(0)
            process(0)

            @pl.when(k2 < n_pairs - 1)
            def _():
                issue(0, off0 + 2 * C2)

            drain(1)
            process(1)
            return 0

        lax.fori_loop(0, n_pairs, pair_body, 0)
        pltpu.sync_copy(acc_v, out_hbm.at[wid])


def _mid_body(dp_ref, xT_ref, xsT_ref, dis_ref):
    deg = jnp.sum(dp_ref[...], axis=0, keepdims=True) + 1.0
    dis = lax.rsqrt(deg)
    dis = dis * (1.5 - 0.5 * deg * dis * dis)
    xs = xT_ref[...] * dis
    xsT_ref[...] = jnp.concatenate(
        [xs, jnp.zeros((16 - PERIODS, xs.shape[1]), xs.dtype)], axis=0)
    dis_ref[...] = dis


_mid_call = pl.pallas_call(
    _mid_body,
    grid=(25,),
    in_specs=[
        pl.BlockSpec((32, BN), lambda i: (0, i)),
        pl.BlockSpec((PERIODS, BN), lambda i: (0, i)),
    ],
    out_specs=[
        pl.BlockSpec((16, BN), lambda i: (0, i)),
        pl.BlockSpec((1, BN), lambda i: (0, i)),
    ],
    out_shape=[
        jax.ShapeDtypeStruct((16, N), jnp.float32),
        jax.ShapeDtypeStruct((1, N), jnp.float32),
    ],
)


def _final_body(gp_ref, xsT_ref, dis_ref, p_ref, q_ref, out_ref):
    dis = dis_ref[...]
    az = p_ref[:, 0:1]
    cz = p_ref[:, 1:2]
    ah = p_ref[:, 2:3]
    ch = p_ref[:, 3:4]
    lw = p_ref[:, 4:5]
    acc = jnp.zeros((HID, dis.shape[1]), jnp.float32)
    for t in range(PERIODS):
        g = (gp_ref[2 * t:2 * t + 1, :] + gp_ref[2 * t + 1:2 * t + 2, :]
             + xsT_ref[t:t + 1, :]) * dis
        u = az * g + cz
        v = ah * g + ch
        acc = acc + q_ref[t:t + 1, 0:1] * (jax.nn.sigmoid(-u) * jnp.tanh(v))
    h = jnp.maximum(acc, 0.0)
    out_ref[...] = jnp.sum(h * lw, axis=0, keepdims=True) + q_ref[12:13, 0:1]


_final_call = pl.pallas_call(
    _final_body,
    grid=(25,),
    in_specs=[
        pl.BlockSpec((24, BN), lambda i: (0, i)),
        pl.BlockSpec((16, BN), lambda i: (0, i)),
        pl.BlockSpec((1, BN), lambda i: (0, i)),
        pl.BlockSpec((HID, 8), lambda i: (0, 0)),
        pl.BlockSpec((16, 8), lambda i: (0, 0)),
    ],
    out_specs=pl.BlockSpec((1, BN), lambda i: (0, i)),
    out_shape=jax.ShapeDtypeStruct((1, N), jnp.float32),
)


def kernel(x, edge_index, edge_weight, att, Wz, bz, LzW, Lzb,
           Wr, br, LrW, Lrb, Wh, bh, LhW, Lhb, linW, linb):
    del Wr, br, LrW, Lrb
    src = edge_index[0].astype(jnp.int32)
    dst = edge_index[1].astype(jnp.int32)
    ew = edge_weight.astype(jnp.float32)
    pk = jnp.bitwise_or(src, jnp.left_shift(dst, 16))
    pad = E_PAD - E
    pk_p = jnp.concatenate([pk, jnp.zeros((pad,), jnp.int32)])
    w_p = jnp.concatenate([ew, jnp.zeros((pad,), jnp.float32)])
    xT = x.T

    deg_part = _deg_kernel(pk_p, w_p)
    xsT, disR = _mid_call(deg_part, xT)
    gpart = _scatter_kernel(xsT, pk_p, w_p)

    top = LzW[:HID]
    az = (Wz @ top)[0]
    cz = bz @ top + Lzb
    toph = LhW[:HID]
    ah = (Wh @ toph)[0]
    ch = bh @ toph + Lhb
    zeros = jnp.zeros((HID,), jnp.float32)
    p_arr = jnp.stack([az, cz, ah, ch, linW[:, 0], zeros, zeros, zeros], axis=1)
    probs = jax.nn.softmax(att)
    q_arr = (jnp.zeros((16, 8), jnp.float32)
             .at[:PERIODS, 0].set(probs)
             .at[12, 0].set(linb[0]))

    out_row = _final_call(gpart, xsT, disR, p_arr, q_arr)
    return out_row.reshape(N, 1)
